# baseline (speedup 1.0000x reference)
import functools

import jax
import jax.numpy as jnp
from jax import lax
from jax.experimental import pallas as pl
from jax.experimental.pallas import tpu as pltpu
from jax.experimental.pallas import tpu_sc as plsc

GAMMA_ = 1.05

N_POINTS = 131072
N_BOXES = 256
K_SLOTS = 128
NFEAT = 5
L = 16
NC = 2
NS = 16
NW = NC * NS
BOXES_PER_W = N_BOXES // NW
CHUNK = 2048
NCHUNKS = N_POINTS // CHUNK
BLOCK_VREGS = 16
BLOCK = BLOCK_VREGS * L
BLOCKS = CHUNK // BLOCK
TBL_W = 8
IDXBUF = K_SLOTS + BLOCK
PARAM_PAD = N_BOXES + L


def _sc_body(p0_hbm, p1_hbm, p2_hbm, p3_hbm, p4_hbm,
             cx_hbm, cy_hbm, t_hbm, out_hbm, tbla_hbm, tblb_hbm,
             ptmp, spxy, buf0, buf1, tb8,
             cxv, cyv, tv, idxv, idxg, rows, cnt_s, infl_s, qctr, done_s,
             sem0, sem1):
    cid = lax.axis_index("c")
    sid = lax.axis_index("s")
    wid = sid * NC + cid

    pltpu.sync_copy(cx_hbm, cxv)
    pltpu.sync_copy(cy_hbm, cyv)
    pltpu.sync_copy(t_hbm, tv)

    lanes = lax.iota(jnp.int32, L)
    zero_v = jnp.zeros((L,), jnp.int32)
    one_v = jnp.full((L,), 1, jnp.int32)

    @pl.when(sid == 0)
    def _():
        qctr[0] = jnp.int32(0)

    planes = (p0_hbm, p1_hbm, p2_hbm, p3_hbm, p4_hbm)
    for half in range(NCHUNKS // NS):
        seg = sid * (NCHUNKS // NS) + half
        pltpu.sync_copy(p0_hbm.at[pl.ds(seg * CHUNK, CHUNK)], spxy.at[seg, 0])
        pltpu.sync_copy(p1_hbm.at[pl.ds(seg * CHUNK, CHUNK)], spxy.at[seg, 1])
        for c in range(NFEAT):
            pltpu.sync_copy(planes[c].at[pl.ds(seg * CHUNK, CHUNK)], ptmp)

            def dein(v, _c, c=c):
                rl = v * L + lanes
                vc = ptmp[pl.ds(v * L, L)]
                plsc.store_scatter(tb8, [rl, zero_v + c], vc)
                return _c

            lax.fori_loop(0, CHUNK // L, dein, 0)

        @pl.when(cid == 0)
        def _(seg=seg):
            pltpu.sync_copy(tb8, tbla_hbm.at[pl.ds(seg * CHUNK, CHUNK)])

        @pl.when(cid == 1)
        def _(seg=seg):
            pltpu.sync_copy(tb8, tblb_hbm.at[pl.ds(seg * CHUNK, CHUNK)])

    plsc.subcore_barrier()

    bufs = (buf0, buf1)
    sems = (sem0, sem1)

    def start(c, parity):
        src = spxy.at[c]
        if parity == 0:
            pltpu.async_copy(src, bufs[0], sems[0])
        else:
            pltpu.async_copy(src, bufs[1], sems[1])

    def wait(c, parity):
        src = spxy.at[c]
        pltpu.make_async_copy(src, bufs[parity], sems[parity]).wait()

    nbox_sc = N_BOXES // NC
    zf = jnp.zeros((L,), jnp.float32)

    def process_box(n):
        pbox = cid * nbox_sc + n
        cx = cxv[pl.ds(pbox, L)][0]
        cy = cyv[pl.ds(pbox, L)][0]
        t = tv[pl.ds(pbox, L)][0]

        def init(kk, _c):
            idxv[pl.ds(kk * L, L)] = zero_v
            return _c

        lax.fori_loop(0, IDXBUF // L, init, 0)
        cnt_s[0] = jnp.int32(0)

        start(jnp.int32(0), 0)
        infl_s[0] = jnp.int32(0)

        def chunk_step(c, parity):
            bufc = bufs[parity]
            active = cnt_s[0] < K_SLOTS

            @pl.when(jnp.logical_and(active, infl_s[0] == c))
            def _():
                @pl.when(c + 1 < NCHUNKS)
                def _():
                    start(c + 1, 1 - parity)

                wait(c, parity)
                infl_s[0] = c + 1
                base = c * CHUNK
                qstop = K_SLOTS - 1

                def vcond(vc):
                    blk, acc = vc
                    return jnp.logical_and(blk < BLOCKS, acc[0] < qstop)

                def vbody(vc):
                    blk, acc = vc
                    for k in range(BLOCK_VREGS):
                        o = blk * BLOCK + k * L
                        rlanes = o + lanes
                        xs = bufc[0, pl.ds(o, L)]
                        ys = bufc[1, pl.ds(o, L)]
                        dx = xs - cx
                        dy = ys - cy
                        d2 = dx * dx + dy * dy
                        m = d2 <= t
                        mi = jnp.where(m, one_v, zero_v)
                        s = plsc.cumsum(mi)
                        pos = acc + s
                        idx = base + rlanes
                        plsc.store_scatter(idxv, [pos], idx, mask=m)
                        acc = acc + plsc.all_reduce_population_count(m)
                    return (blk + 1, acc)

                acc0 = zero_v + (cnt_s[0] - 1)
                _, acc = lax.while_loop(vcond, vbody, (jnp.int32(0), acc0))
                cnt_s[0] = acc[0] + 1

            @pl.when(jnp.logical_and(jnp.logical_not(active), infl_s[0] == c))
            def _():
                wait(c, parity)
                infl_s[0] = jnp.int32(-1)

        def chunk_body(c2, _c):
            chunk_step(2 * c2, 0)
            chunk_step(2 * c2 + 1, 1)
            return _c

        lax.fori_loop(0, NCHUNKS // 2, chunk_body, 0)

        def cp(kk, _c):
            idxg[pl.ds(kk * L, L)] = idxv[pl.ds(kk * L, L)]
            return _c

        lax.fori_loop(0, K_SLOTS // L, cp, 0)

        @pl.when(cid == 0)
        def _():
            pltpu.async_copy(tbla_hbm.at[idxg], rows, sem0).wait()

        @pl.when(cid == 1)
        def _():
            pltpu.async_copy(tblb_hbm.at[idxg], rows, sem0).wait()

        cnt = cnt_s[0]

        @pl.when(cnt < K_SLOTS)
        def _():
            cstop = zero_v + cnt * TBL_W

            def zbody(g, _c):
                flat = g * L + lanes
                mz = flat >= cstop
                row = lax.shift_right_logical(flat, 3)
                col = lax.bitwise_and(flat, jnp.full((L,), TBL_W - 1,
                                                     jnp.int32))
                plsc.store_scatter(rows, [row, col], zf, mask=mz)
                return _c

            lax.fori_loop(0, (K_SLOTS * TBL_W) // L, zbody, 0)

        pltpu.sync_copy(rows.at[:, pl.ds(0, NFEAT)], out_hbm.at[pbox])

    done_s[0] = jnp.int32(0)

    def box_trial(_it, _c):
        @pl.when(done_s[0] == 0)
        def _():
            n = plsc.fetch_and_add(qctr.at[0], 1, subcore_id=0)

            @pl.when(n >= nbox_sc)
            def _():
                done_s[0] = jnp.int32(1)

            @pl.when(n < nbox_sc)
            def _():
                process_box(n)

        return _c

    lax.fori_loop(0, nbox_sc, box_trial, 0)


@functools.partial(
    pl.kernel,
    out_type=(
        jax.ShapeDtypeStruct((N_BOXES, K_SLOTS, NFEAT), jnp.float32),
        jax.ShapeDtypeStruct((N_POINTS, TBL_W), jnp.float32),
        jax.ShapeDtypeStruct((N_POINTS, TBL_W), jnp.float32),
    ),
    mesh=plsc.VectorSubcoreMesh(core_axis_name="c", subcore_axis_name="s"),
    scratch_types=[
        pltpu.VMEM((CHUNK,), jnp.float32),
        pltpu.VMEM_SHARED((NCHUNKS, 2, CHUNK), jnp.float32),
        pltpu.VMEM((2, CHUNK), jnp.float32),
        pltpu.VMEM((2, CHUNK), jnp.float32),
        pltpu.VMEM((CHUNK, TBL_W), jnp.float32),
        pltpu.VMEM((PARAM_PAD,), jnp.float32),
        pltpu.VMEM((PARAM_PAD,), jnp.float32),
        pltpu.VMEM((PARAM_PAD,), jnp.float32),
        pltpu.VMEM((IDXBUF,), jnp.int32),
        pltpu.VMEM((K_SLOTS,), jnp.int32),
        pltpu.VMEM((K_SLOTS, TBL_W), jnp.float32),
        pltpu.SMEM((1,), jnp.int32),
        pltpu.SMEM((1,), jnp.int32),
        pltpu.SMEM((1,), jnp.int32),
        pltpu.SMEM((1,), jnp.int32),
        pltpu.SemaphoreType.DMA,
        pltpu.SemaphoreType.DMA,
    ],
    compiler_params=pltpu.CompilerParams(
        needs_layout_passes=False, use_tc_tiling_on_sc=False
    ),
)
def _voxel_sample_sc(p0, p1, p2, p3, p4, cx_hbm, cy_hbm, t_hbm, out_hbm,
                     tbla_hbm, tblb_hbm, *scratch):
    _sc_body(p0, p1, p2, p3, p4, cx_hbm, cy_hbm, t_hbm, out_hbm,
             tbla_hbm, tblb_hbm, *scratch)


def _depad_body(in_ref, x_ref, y_ref, z_ref, w_ref, v_ref):
    blk = in_ref[...]
    x_ref[...] = blk[:, 0]
    y_ref[...] = blk[:, 1]
    z_ref[...] = blk[:, 2]
    w_ref[...] = blk[:, 3]
    v_ref[...] = blk[:, 4]


_DEPAD_BLK = 8192
_depad_tc = pl.pallas_call(
    _depad_body,
    grid=(N_POINTS // _DEPAD_BLK,),
    in_specs=[pl.BlockSpec((_DEPAD_BLK, NFEAT), lambda i: (i, 0))],
    out_specs=[pl.BlockSpec((_DEPAD_BLK,), lambda i: (i,))] * NFEAT,
    out_shape=[jax.ShapeDtypeStruct((N_POINTS,), jnp.float32)] * NFEAT,
)


def _squared_threshold(r):
    t = r * r
    neg_inf = jnp.float32(-jnp.inf)
    pos_inf = jnp.float32(jnp.inf)
    for _ in range(8):
        t = jnp.where(jnp.sqrt(t) > r, jnp.nextafter(t, neg_inf), t)
    for _ in range(8):
        tn = jnp.nextafter(t, pos_inf)
        t = jnp.where(jnp.sqrt(tn) <= r, tn, t)
    return t


def kernel(cur_points, cur_boxes, num_sample):
    del num_sample
    p0, p1, p2, p3, p4 = _depad_tc(cur_points.astype(jnp.float32))
    r = jnp.linalg.norm(cur_boxes[:, 3:5] / 2.0, axis=-1) * GAMMA_
    t = _squared_threshold(r.astype(jnp.float32))
    pad = jnp.zeros((PARAM_PAD - N_BOXES,), jnp.float32)
    cx = jnp.concatenate([cur_boxes[:, 0].astype(jnp.float32), pad])
    cy = jnp.concatenate([cur_boxes[:, 1].astype(jnp.float32), pad])
    t = jnp.concatenate([t, pad])
    out, _, _ = _voxel_sample_sc(p0, p1, p2, p3, p4, cx, cy, t)
    return out


# speedup vs baseline: 1.2545x; 1.2545x over previous
import functools

import jax
import jax.numpy as jnp
from jax import lax
from jax.experimental import pallas as pl
from jax.experimental.pallas import tpu as pltpu
from jax.experimental.pallas import tpu_sc as plsc

GAMMA_ = 1.05

N_POINTS = 131072
N_BOXES = 256
K_SLOTS = 128
NFEAT = 5
L = 16
NC = 2
NS = 16
NW = NC * NS
BOXES_PER_W = N_BOXES // NW
CHUNK = 2048
NCHUNKS = N_POINTS // CHUNK
BLOCK_VREGS = 16
BLOCK = BLOCK_VREGS * L
BLOCKS = CHUNK // BLOCK
TBL_W = 8
IDXBUF = K_SLOTS + BLOCK
PARAM_PAD = N_BOXES + L


def _sc_body(flat_hbm, cx_hbm, cy_hbm, t_hbm, out_hbm, tbla_hbm, tblb_hbm,
             raw5, spxy, buf0, buf1, xtmp, ytmp, tb8,
             cxv, cyv, tv, idxv, idxg, rows, cnt_s, infl_s, qctr, done_s,
             sem0, sem1):
    cid = lax.axis_index("c")
    sid = lax.axis_index("s")
    wid = sid * NC + cid

    pltpu.sync_copy(cx_hbm, cxv)
    pltpu.sync_copy(cy_hbm, cyv)
    pltpu.sync_copy(t_hbm, tv)

    lanes = lax.iota(jnp.int32, L)
    zero_v = jnp.zeros((L,), jnp.int32)
    one_v = jnp.full((L,), 1, jnp.int32)

    @pl.when(sid == 0)
    def _():
        qctr[0] = jnp.int32(0)

    for half in range(NCHUNKS // NS):
        seg = sid * (NCHUNKS // NS) + half
        pltpu.sync_copy(
            flat_hbm.at[pl.ds(seg * CHUNK * NFEAT, CHUNK * NFEAT)], raw5)

        def dein(v, _c):
            rl = v * L + lanes
            rl5 = rl * NFEAT
            vals = []
            for c in range(NFEAT):
                vc = plsc.load_gather(raw5, [rl5 + c])
                plsc.store_scatter(tb8, [rl, zero_v + c], vc)
                vals.append(vc)
            xtmp[pl.ds(v * L, L)] = vals[0]
            ytmp[pl.ds(v * L, L)] = vals[1]
            return _c

        lax.fori_loop(0, CHUNK // L, dein, 0)
        pltpu.sync_copy(xtmp, spxy.at[seg, 0])
        pltpu.sync_copy(ytmp, spxy.at[seg, 1])

        @pl.when(cid == 0)
        def _(seg=seg):
            pltpu.sync_copy(tb8, tbla_hbm.at[pl.ds(seg * CHUNK, CHUNK)])

        @pl.when(cid == 1)
        def _(seg=seg):
            pltpu.sync_copy(tb8, tblb_hbm.at[pl.ds(seg * CHUNK, CHUNK)])

    plsc.subcore_barrier()

    bufs = (buf0, buf1)
    sems = (sem0, sem1)

    def start(c, parity):
        src = spxy.at[c]
        if parity == 0:
            pltpu.async_copy(src, bufs[0], sems[0])
        else:
            pltpu.async_copy(src, bufs[1], sems[1])

    def wait(c, parity):
        src = spxy.at[c]
        pltpu.make_async_copy(src, bufs[parity], sems[parity]).wait()

    nbox_sc = N_BOXES // NC
    zf = jnp.zeros((L,), jnp.float32)

    def process_box(n):
        pbox = cid * nbox_sc + n
        cx = cxv[pl.ds(pbox, L)][0]
        cy = cyv[pl.ds(pbox, L)][0]
        t = tv[pl.ds(pbox, L)][0]

        def init(kk, _c):
            idxv[pl.ds(kk * L, L)] = zero_v
            return _c

        lax.fori_loop(0, IDXBUF // L, init, 0)
        cnt_s[0] = jnp.int32(0)

        start(jnp.int32(0), 0)
        infl_s[0] = jnp.int32(0)

        def chunk_step(c, parity):
            bufc = bufs[parity]
            active = cnt_s[0] < K_SLOTS

            @pl.when(jnp.logical_and(active, infl_s[0] == c))
            def _():
                @pl.when(c + 1 < NCHUNKS)
                def _():
                    start(c + 1, 1 - parity)

                wait(c, parity)
                infl_s[0] = c + 1
                base = c * CHUNK
                qstop = K_SLOTS - 1

                def vcond(vc):
                    blk, acc = vc
                    return jnp.logical_and(blk < BLOCKS, acc[0] < qstop)

                def vbody(vc):
                    blk, acc = vc
                    for k in range(BLOCK_VREGS):
                        o = blk * BLOCK + k * L
                        rlanes = o + lanes
                        xs = bufc[0, pl.ds(o, L)]
                        ys = bufc[1, pl.ds(o, L)]
                        dx = xs - cx
                        dy = ys - cy
                        d2 = dx * dx + dy * dy
                        m = d2 <= t
                        mi = jnp.where(m, one_v, zero_v)
                        s = plsc.cumsum(mi)
                        pos = acc + s
                        idx = base + rlanes
                        plsc.store_scatter(idxv, [pos], idx, mask=m)
                        acc = acc + plsc.all_reduce_population_count(m)
                    return (blk + 1, acc)

                acc0 = zero_v + (cnt_s[0] - 1)
                _, acc = lax.while_loop(vcond, vbody, (jnp.int32(0), acc0))
                cnt_s[0] = acc[0] + 1

            @pl.when(jnp.logical_and(jnp.logical_not(active), infl_s[0] == c))
            def _():
                wait(c, parity)
                infl_s[0] = jnp.int32(-1)

        def chunk_body(c2, _c):
            chunk_step(2 * c2, 0)
            chunk_step(2 * c2 + 1, 1)
            return _c

        lax.fori_loop(0, NCHUNKS // 2, chunk_body, 0)

        def cp(kk, _c):
            idxg[pl.ds(kk * L, L)] = idxv[pl.ds(kk * L, L)]
            return _c

        lax.fori_loop(0, K_SLOTS // L, cp, 0)

        @pl.when(cid == 0)
        def _():
            pltpu.async_copy(tbla_hbm.at[idxg], rows, sem0).wait()

        @pl.when(cid == 1)
        def _():
            pltpu.async_copy(tblb_hbm.at[idxg], rows, sem0).wait()

        cnt = cnt_s[0]

        @pl.when(cnt < K_SLOTS)
        def _():
            cstop = zero_v + cnt * TBL_W

            def zbody(g, _c):
                flat = g * L + lanes
                mz = flat >= cstop
                row = lax.shift_right_logical(flat, 3)
                col = lax.bitwise_and(flat, jnp.full((L,), TBL_W - 1,
                                                     jnp.int32))
                plsc.store_scatter(rows, [row, col], zf, mask=mz)
                return _c

            lax.fori_loop(0, (K_SLOTS * TBL_W) // L, zbody, 0)

        pltpu.sync_copy(rows.at[:, pl.ds(0, NFEAT)], out_hbm.at[pbox])

    done_s[0] = jnp.int32(0)

    def box_trial(_it, _c):
        @pl.when(done_s[0] == 0)
        def _():
            n = plsc.fetch_and_add(qctr.at[0], 1, subcore_id=0)

            @pl.when(n >= nbox_sc)
            def _():
                done_s[0] = jnp.int32(1)

            @pl.when(n < nbox_sc)
            def _():
                process_box(n)

        return _c

    lax.fori_loop(0, nbox_sc, box_trial, 0)


@functools.partial(
    pl.kernel,
    out_type=(
        jax.ShapeDtypeStruct((N_BOXES, K_SLOTS, NFEAT), jnp.float32),
        jax.ShapeDtypeStruct((N_POINTS, TBL_W), jnp.float32),
        jax.ShapeDtypeStruct((N_POINTS, TBL_W), jnp.float32),
    ),
    mesh=plsc.VectorSubcoreMesh(core_axis_name="c", subcore_axis_name="s"),
    scratch_types=[
        pltpu.VMEM((CHUNK * NFEAT,), jnp.float32),
        pltpu.VMEM_SHARED((NCHUNKS, 2, CHUNK), jnp.float32),
        pltpu.VMEM((2, CHUNK), jnp.float32),
        pltpu.VMEM((2, CHUNK), jnp.float32),
        pltpu.VMEM((CHUNK,), jnp.float32),
        pltpu.VMEM((CHUNK,), jnp.float32),
        pltpu.VMEM((CHUNK, TBL_W), jnp.float32),
        pltpu.VMEM((PARAM_PAD,), jnp.float32),
        pltpu.VMEM((PARAM_PAD,), jnp.float32),
        pltpu.VMEM((PARAM_PAD,), jnp.float32),
        pltpu.VMEM((IDXBUF,), jnp.int32),
        pltpu.VMEM((K_SLOTS,), jnp.int32),
        pltpu.VMEM((K_SLOTS, TBL_W), jnp.float32),
        pltpu.SMEM((1,), jnp.int32),
        pltpu.SMEM((1,), jnp.int32),
        pltpu.SMEM((1,), jnp.int32),
        pltpu.SMEM((1,), jnp.int32),
        pltpu.SemaphoreType.DMA,
        pltpu.SemaphoreType.DMA,
    ],
    compiler_params=pltpu.CompilerParams(
        needs_layout_passes=False, use_tc_tiling_on_sc=False
    ),
)
def _voxel_sample_sc(flat_hbm, cx_hbm, cy_hbm, t_hbm, out_hbm,
                     tbla_hbm, tblb_hbm, *scratch):
    _sc_body(flat_hbm, cx_hbm, cy_hbm, t_hbm, out_hbm, tbla_hbm, tblb_hbm,
             *scratch)


def _squared_threshold(r):
    t = r * r
    neg_inf = jnp.float32(-jnp.inf)
    pos_inf = jnp.float32(jnp.inf)
    for _ in range(8):
        t = jnp.where(jnp.sqrt(t) > r, jnp.nextafter(t, neg_inf), t)
    for _ in range(8):
        tn = jnp.nextafter(t, pos_inf)
        t = jnp.where(jnp.sqrt(tn) <= r, tn, t)
    return t


def kernel(cur_points, cur_boxes, num_sample):
    del num_sample
    flat = jnp.reshape(cur_points.astype(jnp.float32), (N_POINTS * NFEAT,))
    r = jnp.linalg.norm(cur_boxes[:, 3:5] / 2.0, axis=-1) * GAMMA_
    t = _squared_threshold(r.astype(jnp.float32))
    pad = jnp.zeros((PARAM_PAD - N_BOXES,), jnp.float32)
    cx = jnp.concatenate([cur_boxes[:, 0].astype(jnp.float32), pad])
    cy = jnp.concatenate([cur_boxes[:, 1].astype(jnp.float32), pad])
    t = jnp.concatenate([t, pad])
    out, _, _ = _voxel_sample_sc(flat, cx, cy, t)
    return out


# speedup vs baseline: 1.2738x; 1.0154x over previous
import functools

import jax
import jax.numpy as jnp
from jax import lax
from jax.experimental import pallas as pl
from jax.experimental.pallas import tpu as pltpu
from jax.experimental.pallas import tpu_sc as plsc

GAMMA_ = 1.05

N_POINTS = 131072
N_BOXES = 256
K_SLOTS = 128
NFEAT = 5
L = 16
NC = 2
NS = 16
NW = NC * NS
BOXES_PER_W = N_BOXES // NW
CHUNK = 4096
NCHUNKS = N_POINTS // CHUNK
BLOCK_VREGS = 16
BLOCK = BLOCK_VREGS * L
BLOCKS = CHUNK // BLOCK
TBL_W = 8
IDXBUF = K_SLOTS + BLOCK
PARAM_PAD = N_BOXES + L


def _sc_body(flat_hbm, cx_hbm, cy_hbm, t_hbm, out_hbm, tbla_hbm, tblb_hbm,
             raw5, spxy, buf0, buf1, xtmp, ytmp, tb8,
             cxv, cyv, tv, idxv, idxg, rows, cnt_s, infl_s, qctr, done_s,
             sem0, sem1):
    cid = lax.axis_index("c")
    sid = lax.axis_index("s")
    wid = sid * NC + cid

    pltpu.sync_copy(cx_hbm, cxv)
    pltpu.sync_copy(cy_hbm, cyv)
    pltpu.sync_copy(t_hbm, tv)

    lanes = lax.iota(jnp.int32, L)
    zero_v = jnp.zeros((L,), jnp.int32)
    one_v = jnp.full((L,), 1, jnp.int32)

    @pl.when(sid == 0)
    def _():
        qctr[0] = jnp.int32(0)

    for half in range(NCHUNKS // NS):
        seg = sid * (NCHUNKS // NS) + half
        pltpu.sync_copy(
            flat_hbm.at[pl.ds(seg * CHUNK * NFEAT, CHUNK * NFEAT)], raw5)

        def dein(v, _c):
            rl = v * L + lanes
            rl5 = rl * NFEAT
            vals = []
            for c in range(NFEAT):
                vc = plsc.load_gather(raw5, [rl5 + c])
                plsc.store_scatter(tb8, [rl, zero_v + c], vc)
                vals.append(vc)
            xtmp[pl.ds(v * L, L)] = vals[0]
            ytmp[pl.ds(v * L, L)] = vals[1]
            return _c

        lax.fori_loop(0, CHUNK // L, dein, 0)
        pltpu.sync_copy(xtmp, spxy.at[seg, 0])
        pltpu.sync_copy(ytmp, spxy.at[seg, 1])

        @pl.when(cid == 0)
        def _(seg=seg):
            pltpu.sync_copy(tb8, tbla_hbm.at[pl.ds(seg * CHUNK, CHUNK)])

        @pl.when(cid == 1)
        def _(seg=seg):
            pltpu.sync_copy(tb8, tblb_hbm.at[pl.ds(seg * CHUNK, CHUNK)])

    plsc.subcore_barrier()

    bufs = (buf0, buf1)
    sems = (sem0, sem1)

    def start(c, parity):
        src = spxy.at[c]
        if parity == 0:
            pltpu.async_copy(src, bufs[0], sems[0])
        else:
            pltpu.async_copy(src, bufs[1], sems[1])

    def wait(c, parity):
        src = spxy.at[c]
        pltpu.make_async_copy(src, bufs[parity], sems[parity]).wait()

    nbox_sc = N_BOXES // NC
    zf = jnp.zeros((L,), jnp.float32)

    def process_box(n):
        pbox = cid * nbox_sc + n
        cx = cxv[pl.ds(pbox, L)][0]
        cy = cyv[pl.ds(pbox, L)][0]
        t = tv[pl.ds(pbox, L)][0]

        def init(kk, _c):
            idxv[pl.ds(kk * L, L)] = zero_v
            return _c

        lax.fori_loop(0, IDXBUF // L, init, 0)
        cnt_s[0] = jnp.int32(0)

        start(jnp.int32(0), 0)
        infl_s[0] = jnp.int32(0)

        def chunk_step(c, parity):
            bufc = bufs[parity]
            active = cnt_s[0] < K_SLOTS

            @pl.when(jnp.logical_and(active, infl_s[0] == c))
            def _():
                @pl.when(c + 1 < NCHUNKS)
                def _():
                    start(c + 1, 1 - parity)

                wait(c, parity)
                infl_s[0] = c + 1
                base = c * CHUNK
                qstop = K_SLOTS - 1

                def vcond(vc):
                    blk, acc = vc
                    return jnp.logical_and(blk < BLOCKS, acc[0] < qstop)

                def vbody(vc):
                    blk, acc = vc
                    for k in range(BLOCK_VREGS):
                        o = blk * BLOCK + k * L
                        rlanes = o + lanes
                        xs = bufc[0, pl.ds(o, L)]
                        ys = bufc[1, pl.ds(o, L)]
                        dx = xs - cx
                        dy = ys - cy
                        d2 = dx * dx + dy * dy
                        m = d2 <= t
                        mi = jnp.where(m, one_v, zero_v)
                        s = plsc.cumsum(mi)
                        pos = acc + s
                        idx = base + rlanes
                        plsc.store_scatter(idxv, [pos], idx, mask=m)
                        acc = acc + plsc.all_reduce_population_count(m)
                    return (blk + 1, acc)

                acc0 = zero_v + (cnt_s[0] - 1)
                _, acc = lax.while_loop(vcond, vbody, (jnp.int32(0), acc0))
                cnt_s[0] = acc[0] + 1

            @pl.when(jnp.logical_and(jnp.logical_not(active), infl_s[0] == c))
            def _():
                wait(c, parity)
                infl_s[0] = jnp.int32(-1)

        def chunk_body(c2, _c):
            chunk_step(2 * c2, 0)
            chunk_step(2 * c2 + 1, 1)
            return _c

        lax.fori_loop(0, NCHUNKS // 2, chunk_body, 0)

        def cp(kk, _c):
            idxg[pl.ds(kk * L, L)] = idxv[pl.ds(kk * L, L)]
            return _c

        lax.fori_loop(0, K_SLOTS // L, cp, 0)

        @pl.when(cid == 0)
        def _():
            pltpu.async_copy(tbla_hbm.at[idxg], rows, sem0).wait()

        @pl.when(cid == 1)
        def _():
            pltpu.async_copy(tblb_hbm.at[idxg], rows, sem0).wait()

        cnt = cnt_s[0]

        @pl.when(cnt < K_SLOTS)
        def _():
            cstop = zero_v + cnt * TBL_W

            def zbody(g, _c):
                flat = g * L + lanes
                mz = flat >= cstop
                row = lax.shift_right_logical(flat, 3)
                col = lax.bitwise_and(flat, jnp.full((L,), TBL_W - 1,
                                                     jnp.int32))
                plsc.store_scatter(rows, [row, col], zf, mask=mz)
                return _c

            lax.fori_loop(0, (K_SLOTS * TBL_W) // L, zbody, 0)

        pltpu.sync_copy(rows.at[:, pl.ds(0, NFEAT)], out_hbm.at[pbox])

    done_s[0] = jnp.int32(0)

    def box_trial(_it, _c):
        @pl.when(done_s[0] == 0)
        def _():
            n = plsc.fetch_and_add(qctr.at[0], 1, subcore_id=0)

            @pl.when(n >= nbox_sc)
            def _():
                done_s[0] = jnp.int32(1)

            @pl.when(n < nbox_sc)
            def _():
                process_box(n)

        return _c

    lax.fori_loop(0, nbox_sc, box_trial, 0)


@functools.partial(
    pl.kernel,
    out_type=(
        jax.ShapeDtypeStruct((N_BOXES, K_SLOTS, NFEAT), jnp.float32),
        jax.ShapeDtypeStruct((N_POINTS, TBL_W), jnp.float32),
        jax.ShapeDtypeStruct((N_POINTS, TBL_W), jnp.float32),
    ),
    mesh=plsc.VectorSubcoreMesh(core_axis_name="c", subcore_axis_name="s"),
    scratch_types=[
        pltpu.VMEM((CHUNK * NFEAT,), jnp.float32),
        pltpu.VMEM_SHARED((NCHUNKS, 2, CHUNK), jnp.float32),
        pltpu.VMEM((2, CHUNK), jnp.float32),
        pltpu.VMEM((2, CHUNK), jnp.float32),
        pltpu.VMEM((CHUNK,), jnp.float32),
        pltpu.VMEM((CHUNK,), jnp.float32),
        pltpu.VMEM((CHUNK, TBL_W), jnp.float32),
        pltpu.VMEM((PARAM_PAD,), jnp.float32),
        pltpu.VMEM((PARAM_PAD,), jnp.float32),
        pltpu.VMEM((PARAM_PAD,), jnp.float32),
        pltpu.VMEM((IDXBUF,), jnp.int32),
        pltpu.VMEM((K_SLOTS,), jnp.int32),
        pltpu.VMEM((K_SLOTS, TBL_W), jnp.float32),
        pltpu.SMEM((1,), jnp.int32),
        pltpu.SMEM((1,), jnp.int32),
        pltpu.SMEM((1,), jnp.int32),
        pltpu.SMEM((1,), jnp.int32),
        pltpu.SemaphoreType.DMA,
        pltpu.SemaphoreType.DMA,
    ],
    compiler_params=pltpu.CompilerParams(
        needs_layout_passes=False, use_tc_tiling_on_sc=False
    ),
)
def _voxel_sample_sc(flat_hbm, cx_hbm, cy_hbm, t_hbm, out_hbm,
                     tbla_hbm, tblb_hbm, *scratch):
    _sc_body(flat_hbm, cx_hbm, cy_hbm, t_hbm, out_hbm, tbla_hbm, tblb_hbm,
             *scratch)


def _squared_threshold(r):
    t = r * r
    neg_inf = jnp.float32(-jnp.inf)
    pos_inf = jnp.float32(jnp.inf)
    for _ in range(8):
        t = jnp.where(jnp.sqrt(t) > r, jnp.nextafter(t, neg_inf), t)
    for _ in range(8):
        tn = jnp.nextafter(t, pos_inf)
        t = jnp.where(jnp.sqrt(tn) <= r, tn, t)
    return t


def kernel(cur_points, cur_boxes, num_sample):
    del num_sample
    flat = jnp.reshape(cur_points.astype(jnp.float32), (N_POINTS * NFEAT,))
    r = jnp.linalg.norm(cur_boxes[:, 3:5] / 2.0, axis=-1) * GAMMA_
    t = _squared_threshold(r.astype(jnp.float32))
    pad = jnp.zeros((PARAM_PAD - N_BOXES,), jnp.float32)
    cx = jnp.concatenate([cur_boxes[:, 0].astype(jnp.float32), pad])
    cy = jnp.concatenate([cur_boxes[:, 1].astype(jnp.float32), pad])
    t = jnp.concatenate([t, pad])
    out, _, _ = _voxel_sample_sc(flat, cx, cy, t)
    return out
